# baseline (speedup 1.0000x reference)
import jax
import jax.numpy as jnp
from jax.experimental import pallas as pl

_B, _S, _D = 4096, 200, 64
_G = _B // 2
_K = 32


def _add_body(x_ref, pos_ref, o_ref):
    o_ref[...] = x_ref[...] + pos_ref[...]


def kernel(x, pos_emb):
    xr = x.reshape(_G, _S, 2 * _D)
    pf = pos_emb.reshape(-1)
    pos2 = jnp.concatenate([pf, pf]).reshape(1, _S, 2 * _D)
    out = pl.pallas_call(
        _add_body,
        grid=(_G // _K,),
        in_specs=[
            pl.BlockSpec((_K, _S, 2 * _D), lambda i: (i, 0, 0)),
            pl.BlockSpec((1, _S, 2 * _D), lambda i: (0, 0, 0)),
        ],
        out_specs=pl.BlockSpec((_K, _S, 2 * _D), lambda i: (i, 0, 0)),
        out_shape=jax.ShapeDtypeStruct((_G, _S, 2 * _D), jnp.float32),
    )(xr, pos2)
    return out.reshape(_B, _S, _D)


# speedup vs baseline: 8.0205x; 8.0205x over previous
import jax
import jax.numpy as jnp
from jax.experimental import pallas as pl

_B, _S, _D = 4096, 200, 64
_SC = 5


def _add_body(x_ref, pos_ref, o_ref):
    o_ref[...] = x_ref[...] + pos_ref[...]


def kernel(x, pos_emb):
    xt = jnp.transpose(x, (1, 2, 0))
    p3 = pos_emb[:, :, None]
    out = pl.pallas_call(
        _add_body,
        grid=(_S // _SC,),
        in_specs=[
            pl.BlockSpec((_SC, _D, _B), lambda i: (i, 0, 0)),
            pl.BlockSpec((_SC, _D, 1), lambda i: (i, 0, 0)),
        ],
        out_specs=pl.BlockSpec((_SC, _D, _B), lambda i: (i, 0, 0)),
        out_shape=jax.ShapeDtypeStruct((_S, _D, _B), jnp.float32),
    )(xt, p3)
    return jnp.transpose(out, (2, 0, 1))


# speedup vs baseline: 8.0593x; 1.0048x over previous
import jax
import jax.numpy as jnp
from jax.experimental import pallas as pl

_B, _S, _D = 4096, 200, 64
_SC = 10


def _add_body(x_ref, pos_ref, o_ref):
    o_ref[...] = x_ref[...] + pos_ref[...]


def kernel(x, pos_emb):
    xt = jnp.transpose(x, (1, 2, 0))
    p3 = pos_emb[:, :, None]
    out = pl.pallas_call(
        _add_body,
        grid=(_S // _SC,),
        in_specs=[
            pl.BlockSpec((_SC, _D, _B), lambda i: (i, 0, 0)),
            pl.BlockSpec((_SC, _D, 1), lambda i: (i, 0, 0)),
        ],
        out_specs=pl.BlockSpec((_SC, _D, _B), lambda i: (i, 0, 0)),
        out_shape=jax.ShapeDtypeStruct((_S, _D, _B), jnp.float32),
    )(xt, p3)
    return jnp.transpose(out, (2, 0, 1))
